# Initial kernel scaffold; baseline (speedup 1.0000x reference)
import functools

import jax
import jax.numpy as jnp
from jax import lax
from jax.experimental import pallas as pl
from jax.experimental.pallas import tpu as pltpu
from jax.experimental.pallas import tpu_sc as plsc

N = 50000
E = 800000
D_IN = 128
D_OUT = 64
DH = 32
K = 10
ALPHA = 0.1

NTILES = 16
LANES = 16

RCH = 25
ROWS_PT = RCH * 128
NPAD = NTILES * ROWS_PT

EB = 49
EROWS_PT = EB * 8
EROWS = NTILES * EROWS_PT
EPAD = EROWS * 128

_mesh = lambda: plsc.VectorSubcoreMesh(core_axis_name="c", subcore_axis_name="s")


def _fill2d(ref, nrows, ncols, val):
    v = jnp.full((LANES,), val, jnp.float32)

    @pl.loop(0, nrows)
    def _(r):
        for j in range(ncols // LANES):
            ref[r, pl.ds(j * LANES, LANES)] = v


def _rsqrt16(d):
    i = plsc.bitcast(d, jnp.int32)
    x = plsc.bitcast(jnp.int32(0x5F3759DF) - (i >> 1), jnp.float32)
    for _ in range(4):
        x = x * (1.5 - 0.5 * d * x * x)
    return x


def _splat16(ref, idx):
    return plsc.load_gather(ref, [jnp.full((LANES,), idx, jnp.int32)])


def _sc_degrees(dst2d):
    @functools.partial(
        pl.kernel,
        out_type=jax.ShapeDtypeStruct((2, NPAD), jnp.float32),
        mesh=_mesh(),
        scratch_types=[
            pltpu.VMEM_SHARED((NPAD, 16), jnp.float32),
            pltpu.VMEM((8, 128), jnp.int32),
            pltpu.VMEM((128, 16), jnp.float32),
            pltpu.VMEM((128, 16), jnp.float32),
            pltpu.VMEM((128, 16), jnp.float32),
            pltpu.VMEM((128,), jnp.float32),
        ],
    )
    def deg_kernel(dst_hbm, dis_hbm, hist, idx_v, ones_v, zeros_v, hb_v, dis_v):
        c = lax.axis_index("c")
        s = lax.axis_index("s")
        rbase = s * ROWS_PT
        ebase = s * EROWS_PT

        _fill2d(ones_v, 128, 16, 1.0)
        _fill2d(zeros_v, 128, 16, 0.0)

        @pl.loop(0, RCH)
        def _(ch):
            pltpu.sync_copy(zeros_v, hist.at[pl.ds(rbase + ch * 128, 128)])

        plsc.subcore_barrier()

        @pl.loop(0, EB)
        def _(b):
            pltpu.sync_copy(dst_hbm.at[pl.ds(ebase + b * 8, 8)], idx_v)
            for j in range(8):
                pltpu.sync_copy(ones_v, hist.at[idx_v.at[j]], add=True)

        plsc.subcore_barrier()

        lane = lax.iota(jnp.int32, 16)

        @pl.loop(0, RCH)
        def _(ch):
            r0 = rbase + ch * 128
            pltpu.sync_copy(hist.at[pl.ds(r0, 128)], hb_v)
            for g in range(8):
                rows = jnp.full((LANES,), g * 16, jnp.int32) + lane
                cnt = plsc.load_gather(hb_v, [rows, jnp.zeros((LANES,), jnp.int32)])
                dis_v[pl.ds(g * 16, 16)] = _rsqrt16(cnt + 1.0)
            pltpu.sync_copy(dis_v, dis_hbm.at[c, pl.ds(r0, 128)])

    return deg_kernel(dst2d)


def _sc_propagate(src2d, dst2d, dis, h2):
    @functools.partial(
        pl.kernel,
        out_type=(
            jax.ShapeDtypeStruct((2, NPAD, DH), jnp.float32),
            jax.ShapeDtypeStruct((2, NPAD, DH), jnp.float32),
        ),
        mesh=_mesh(),
        scratch_types=[
            pltpu.VMEM_SHARED((NPAD, DH), jnp.float32),
            pltpu.VMEM((8, 128), jnp.int32),
            pltpu.VMEM((8, 128), jnp.int32),
            pltpu.VMEM((128, DH), jnp.float32),
            pltpu.VMEM((128, DH), jnp.float32),
            pltpu.VMEM((128, DH), jnp.float32),
            pltpu.VMEM((128, DH), jnp.float32),
            pltpu.VMEM((128, DH), jnp.float32),
            pltpu.VMEM((128, DH), jnp.float32),
            pltpu.VMEM((ROWS_PT,), jnp.float32),
            pltpu.SemaphoreType.DMA,
        ],
    )
    def prop_kernel(src_hbm, dst_hbm, dis_hbm, h_hbm, y_hbm, z_hbm,
                    agg, isrc_v, idst_v, g_v, a_v, y_v, h_v, o_v, zeros_v,
                    dis_l, sem):
        c = lax.axis_index("c")
        s = lax.axis_index("s")
        rbase = s * ROWS_PT
        ebase = s * EROWS_PT

        y_c = y_hbm.at[c]
        z_c = z_hbm.at[c]
        h_c = h_hbm.at[c]

        _fill2d(zeros_v, 128, DH, 0.0)
        pltpu.sync_copy(dis_hbm.at[c, pl.ds(rbase, ROWS_PT)], dis_l)

        @pl.loop(0, RCH)
        def _(ch):
            pltpu.sync_copy(zeros_v, agg.at[pl.ds(rbase + ch * 128, 128)])

        @pl.loop(0, RCH)
        def _(ch):
            r0 = rbase + ch * 128
            pltpu.sync_copy(h_c.at[pl.ds(r0, 128)], h_v)

            @pl.loop(0, 128)
            def _(r):
                dv = _splat16(dis_l, ch * 128 + r)
                for cc in range(DH // LANES):
                    sl = pl.ds(cc * LANES, LANES)
                    o_v[r, sl] = dv * h_v[r, sl]

            pltpu.sync_copy(o_v, y_c.at[pl.ds(r0, 128)])

        plsc.subcore_barrier()

        def scatter_phase():
            @pl.loop(0, EB)
            def _(b):
                e0 = ebase + b * 8
                pltpu.sync_copy(src_hbm.at[pl.ds(e0, 8)], isrc_v)
                pltpu.sync_copy(dst_hbm.at[pl.ds(e0, 8)], idst_v)
                for j in range(8):
                    pltpu.async_copy(y_c.at[isrc_v.at[j]], g_v, sem).wait()
                    pltpu.sync_copy(g_v, agg.at[idst_v.at[j]], add=True)

        def dense_phase(last):
            @pl.loop(0, RCH)
            def _(ch):
                r0 = rbase + ch * 128
                pltpu.sync_copy(agg.at[pl.ds(r0, 128)], a_v)
                pltpu.sync_copy(zeros_v, agg.at[pl.ds(r0, 128)])
                pltpu.sync_copy(y_c.at[pl.ds(r0, 128)], y_v)
                pltpu.sync_copy(h_c.at[pl.ds(r0, 128)], h_v)

                @pl.loop(0, 128)
                def _(r):
                    dv = _splat16(dis_l, ch * 128 + r)
                    c09 = (1.0 - ALPHA) * dv
                    for cc in range(DH // LANES):
                        sl = pl.ds(cc * LANES, LANES)
                        v = a_v[r, sl] + y_v[r, sl]
                        z = c09 * v + ALPHA * h_v[r, sl]
                        o_v[r, sl] = z if last else dv * z

                if last:
                    pltpu.sync_copy(o_v, z_c.at[pl.ds(r0, 128)])
                else:
                    pltpu.sync_copy(o_v, y_c.at[pl.ds(r0, 128)])

        @pl.loop(0, K - 1)
        def _(k):
            scatter_phase()
            plsc.subcore_barrier()
            dense_phase(False)
            plsc.subcore_barrier()

        scatter_phase()
        plsc.subcore_barrier()
        dense_phase(True)

    return prop_kernel(src2d, dst2d, dis, h2)


_MLP_BLK = 1024


def _mlp_body(x_ref, w1_ref, b1_ref, w2_ref, b2_ref, o_ref):
    hid = jnp.dot(x_ref[...], w1_ref[...], preferred_element_type=jnp.float32)
    hid = jnp.maximum(hid + b1_ref[...], 0.0)
    out = jnp.dot(hid, w2_ref[...], preferred_element_type=jnp.float32)
    o_ref[...] = out + b2_ref[...]


def _tc_mlp(xp, W1, b1, W2, b2):
    grid = NPAD // _MLP_BLK
    return pl.pallas_call(
        _mlp_body,
        grid=(grid,),
        in_specs=[
            pl.BlockSpec((_MLP_BLK, D_IN), lambda i: (i, 0)),
            pl.BlockSpec((D_IN, D_IN), lambda i: (0, 0)),
            pl.BlockSpec((1, D_IN), lambda i: (0, 0)),
            pl.BlockSpec((D_IN, D_OUT), lambda i: (0, 0)),
            pl.BlockSpec((1, D_OUT), lambda i: (0, 0)),
        ],
        out_specs=pl.BlockSpec((_MLP_BLK, D_OUT), lambda i: (i, 0)),
        out_shape=jax.ShapeDtypeStruct((NPAD, D_OUT), jnp.float32),
    )(xp, W1, b1.reshape(1, D_IN), W2, b2.reshape(1, D_OUT))


def kernel(x, edge_index, W1, b1, W2, b2):
    src = edge_index[0]
    dst = edge_index[1]
    pad = jnp.full((EPAD - E,), N, jnp.int32)
    src2d = jnp.concatenate([src, pad]).reshape(EROWS, 128)
    dst2d = jnp.concatenate([dst, pad]).reshape(EROWS, 128)
    xp = jnp.pad(x, ((0, NPAD - N), (0, 0)))

    h = _tc_mlp(xp, W1, b1, W2, b2)
    h2 = jnp.stack([h[:, :DH], h[:, DH:]])
    dis = _sc_degrees(dst2d)
    _, z2 = _sc_propagate(src2d, dst2d, dis, h2)
    return jnp.concatenate([z2[0, :N], z2[1, :N]], axis=1)


# speedup vs baseline: 11.0896x; 11.0896x over previous
import dataclasses
import functools

import jax
import jax.numpy as jnp
from jax import lax
from jax.experimental import pallas as pl
from jax.experimental.pallas import tpu as pltpu
from jax.experimental.pallas import tpu_sc as plsc

N = 50000
E = 800000
D_IN = 128
D_OUT = 64
DH = 32
K = 10
ALPHA = 0.1

NTILES = 16
LANES = 16

RCH = 25
ROWS_PT = RCH * 128
NPAD = NTILES * ROWS_PT

EB = 49
EROWS_PT = EB * 8
EROWS = NTILES * EROWS_PT
EPAD = EROWS * 128

_mesh = lambda: plsc.VectorSubcoreMesh(core_axis_name="c", subcore_axis_name="s")


def _sc_params():
    cp = pltpu.CompilerParams()
    fields = pltpu.CompilerParams.__dataclass_fields__
    if "needs_layout_passes" in fields:
        cp = dataclasses.replace(cp, needs_layout_passes=False)
    if "use_tc_tiling_on_sc" in fields:
        cp = dataclasses.replace(cp, use_tc_tiling_on_sc=False)
    return cp


def _fill2d(ref, nrows, ncols, val):
    v = jnp.full((LANES,), val, jnp.float32)

    @pl.loop(0, nrows)
    def _(r):
        for j in range(ncols // LANES):
            ref[r, pl.ds(j * LANES, LANES)] = v


def _rsqrt16(d):
    i = plsc.bitcast(d, jnp.int32)
    x = plsc.bitcast(jnp.int32(0x5F3759DF) - (i >> 1), jnp.float32)
    for _ in range(4):
        x = x * (1.5 - 0.5 * d * x * x)
    return x


def _splat16(ref, idx):
    return plsc.load_gather(ref, [jnp.full((LANES,), idx, jnp.int32)])


def _sc_degrees(dst2d):
    @functools.partial(
        pl.kernel,
        out_type=jax.ShapeDtypeStruct((2, NPAD), jnp.float32),
        mesh=_mesh(),
        compiler_params=_sc_params(),
        scratch_types=[
            pltpu.VMEM_SHARED((NPAD, 16), jnp.float32),
            pltpu.VMEM((8, 128), jnp.int32),
            pltpu.VMEM((128, 16), jnp.float32),
            pltpu.VMEM((128, 16), jnp.float32),
            pltpu.VMEM((128, 16), jnp.float32),
            pltpu.VMEM((128,), jnp.float32),
        ],
    )
    def deg_kernel(dst_hbm, dis_hbm, hist, idx_v, ones_v, zeros_v, hb_v, dis_v):
        c = lax.axis_index("c")
        s = lax.axis_index("s")
        rbase = s * ROWS_PT
        ebase = s * EROWS_PT

        _fill2d(ones_v, 128, 16, 1.0)
        _fill2d(zeros_v, 128, 16, 0.0)

        @pl.loop(0, RCH)
        def _(ch):
            pltpu.sync_copy(zeros_v, hist.at[pl.ds(rbase + ch * 128, 128)])

        plsc.subcore_barrier()

        @pl.loop(0, EB)
        def _(b):
            pltpu.sync_copy(dst_hbm.at[pl.ds(ebase + b * 8, 8)], idx_v)
            for j in range(8):
                pltpu.sync_copy(ones_v, hist.at[idx_v.at[j]], add=True)

        plsc.subcore_barrier()

        lane = lax.iota(jnp.int32, 16)

        @pl.loop(0, RCH)
        def _(ch):
            r0 = rbase + ch * 128
            pltpu.sync_copy(hist.at[pl.ds(r0, 128)], hb_v)
            for g in range(8):
                rows = jnp.full((LANES,), g * 16, jnp.int32) + lane
                cnt = plsc.load_gather(hb_v, [rows, jnp.zeros((LANES,), jnp.int32)])
                dis_v[pl.ds(g * 16, 16)] = _rsqrt16(cnt + 1.0)
            pltpu.sync_copy(dis_v, dis_hbm.at[c, pl.ds(r0, 128)])

    return deg_kernel(dst2d)


def _sc_propagate(src2d, dst2d, dis, h2):
    @functools.partial(
        pl.kernel,
        out_type=(
            jax.ShapeDtypeStruct((2, NPAD, DH), jnp.float32),
            jax.ShapeDtypeStruct((2, NPAD, DH), jnp.float32),
        ),
        mesh=_mesh(),
        compiler_params=_sc_params(),
        scratch_types=[
            pltpu.VMEM_SHARED((NPAD, DH), jnp.float32),
            pltpu.VMEM((8, 128), jnp.int32),
            pltpu.VMEM((8, 128), jnp.int32),
            pltpu.VMEM((128, DH), jnp.float32),
            pltpu.VMEM((128, DH), jnp.float32),
            pltpu.VMEM((128, DH), jnp.float32),
            pltpu.VMEM((128, DH), jnp.float32),
            pltpu.VMEM((128, DH), jnp.float32),
            pltpu.VMEM((128,), jnp.float32),
            pltpu.SemaphoreType.DMA,
        ],
    )
    def prop_kernel(src_hbm, dst_hbm, dis_hbm, h_hbm, y_hbm, z_hbm,
                    agg, isrc_v, idst_v, g_v, a_v, y_v, h_v, zeros_v,
                    dis_c, sem):
        o_v = g_v
        c = lax.axis_index("c")
        s = lax.axis_index("s")
        rbase = s * ROWS_PT
        ebase = s * EROWS_PT

        y_c = y_hbm.at[c]
        z_c = z_hbm.at[c]
        h_c = h_hbm.at[c]

        _fill2d(zeros_v, 128, DH, 0.0)

        @pl.loop(0, RCH)
        def _(ch):
            pltpu.sync_copy(zeros_v, agg.at[pl.ds(rbase + ch * 128, 128)])

        @pl.loop(0, RCH)
        def _(ch):
            r0 = rbase + ch * 128
            pltpu.sync_copy(h_c.at[pl.ds(r0, 128)], h_v)
            pltpu.sync_copy(dis_hbm.at[c, pl.ds(r0, 128)], dis_c)

            @pl.loop(0, 128)
            def _(r):
                dv = _splat16(dis_c, r)
                for cc in range(DH // LANES):
                    sl = pl.ds(cc * LANES, LANES)
                    o_v[r, sl] = dv * h_v[r, sl]

            pltpu.sync_copy(o_v, y_c.at[pl.ds(r0, 128)])

        plsc.subcore_barrier()

        def scatter_phase():
            @pl.loop(0, EB)
            def _(b):
                e0 = ebase + b * 8
                pltpu.sync_copy(src_hbm.at[pl.ds(e0, 8)], isrc_v)
                pltpu.sync_copy(dst_hbm.at[pl.ds(e0, 8)], idst_v)
                for j in range(8):
                    pltpu.async_copy(y_c.at[isrc_v.at[j]], g_v, sem).wait()
                    pltpu.sync_copy(g_v, agg.at[idst_v.at[j]], add=True)

        def dense_phase(last):
            @pl.loop(0, RCH)
            def _(ch):
                r0 = rbase + ch * 128
                pltpu.sync_copy(agg.at[pl.ds(r0, 128)], a_v)
                pltpu.sync_copy(zeros_v, agg.at[pl.ds(r0, 128)])
                pltpu.sync_copy(y_c.at[pl.ds(r0, 128)], y_v)
                pltpu.sync_copy(h_c.at[pl.ds(r0, 128)], h_v)
                pltpu.sync_copy(dis_hbm.at[c, pl.ds(r0, 128)], dis_c)

                @pl.loop(0, 128)
                def _(r):
                    dv = _splat16(dis_c, r)
                    c09 = (1.0 - ALPHA) * dv
                    for cc in range(DH // LANES):
                        sl = pl.ds(cc * LANES, LANES)
                        v = a_v[r, sl] + y_v[r, sl]
                        z = c09 * v + ALPHA * h_v[r, sl]
                        o_v[r, sl] = z if last else dv * z

                if last:
                    pltpu.sync_copy(o_v, z_c.at[pl.ds(r0, 128)])
                else:
                    pltpu.sync_copy(o_v, y_c.at[pl.ds(r0, 128)])

        @pl.loop(0, K - 1)
        def _(k):
            scatter_phase()
            plsc.subcore_barrier()
            dense_phase(False)
            plsc.subcore_barrier()

        scatter_phase()
        plsc.subcore_barrier()
        dense_phase(True)

    return prop_kernel(src2d, dst2d, dis, h2)


_MLP_BLK = 1024


def _mlp_body(x_ref, w1_ref, b1_ref, w2_ref, b2_ref, o_ref):
    hid = jnp.dot(x_ref[...], w1_ref[...], preferred_element_type=jnp.float32)
    hid = jnp.maximum(hid + b1_ref[...], 0.0)
    out = jnp.dot(hid, w2_ref[...], preferred_element_type=jnp.float32)
    o_ref[...] = out + b2_ref[...]


def _tc_mlp(xp, W1, b1, W2, b2):
    grid = NPAD // _MLP_BLK
    return pl.pallas_call(
        _mlp_body,
        grid=(grid,),
        in_specs=[
            pl.BlockSpec((_MLP_BLK, D_IN), lambda i: (i, 0)),
            pl.BlockSpec((D_IN, D_IN), lambda i: (0, 0)),
            pl.BlockSpec((1, D_IN), lambda i: (0, 0)),
            pl.BlockSpec((D_IN, D_OUT), lambda i: (0, 0)),
            pl.BlockSpec((1, D_OUT), lambda i: (0, 0)),
        ],
        out_specs=pl.BlockSpec((_MLP_BLK, D_OUT), lambda i: (i, 0)),
        out_shape=jax.ShapeDtypeStruct((NPAD, D_OUT), jnp.float32),
    )(xp, W1, b1.reshape(1, D_IN), W2, b2.reshape(1, D_OUT))


def kernel(x, edge_index, W1, b1, W2, b2):
    src = edge_index[0]
    dst = edge_index[1]
    pad = jnp.full((EPAD - E,), N, jnp.int32)
    src2d = jnp.concatenate([src, pad]).reshape(EROWS, 128)
    dst2d = jnp.concatenate([dst, pad]).reshape(EROWS, 128)
    xp = jnp.pad(x, ((0, NPAD - N), (0, 0)))

    h = _tc_mlp(xp, W1, b1, W2, b2)
    h2 = jnp.stack([h[:, :DH], h[:, DH:]])
    dis = _sc_degrees(dst2d)
    _, z2 = _sc_propagate(src2d, dst2d, dis, h2)
    return jnp.concatenate([z2[0, :N], z2[1, :N]], axis=1)


# speedup vs baseline: 17.4632x; 1.5747x over previous
import dataclasses
import functools

import jax
import jax.numpy as jnp
from jax import lax
from jax.experimental import pallas as pl
from jax.experimental.pallas import tpu as pltpu
from jax.experimental.pallas import tpu_sc as plsc

N = 50000
E = 800000
D_IN = 128
D_OUT = 64
DH = 32
K = 10
ALPHA = 0.1

NTILES = 16
LANES = 16

RCH = 25
ROWS_PT = RCH * 128
NPAD = NTILES * ROWS_PT

EB = 49
EROWS_PT = EB * 8
EROWS = NTILES * EROWS_PT
EPAD = EROWS * 128

_mesh = lambda: plsc.VectorSubcoreMesh(core_axis_name="c", subcore_axis_name="s")


def _sc_params():
    cp = pltpu.CompilerParams()
    fields = pltpu.CompilerParams.__dataclass_fields__
    if "needs_layout_passes" in fields:
        cp = dataclasses.replace(cp, needs_layout_passes=False)
    if "use_tc_tiling_on_sc" in fields:
        cp = dataclasses.replace(cp, use_tc_tiling_on_sc=False)
    return cp


def _fill2d(ref, nrows, ncols, val):
    v = jnp.full((LANES,), val, jnp.float32)

    @pl.loop(0, nrows)
    def _(r):
        for j in range(ncols // LANES):
            ref[r, pl.ds(j * LANES, LANES)] = v


def _rsqrt16(d):
    i = plsc.bitcast(d, jnp.int32)
    x = plsc.bitcast(jnp.int32(0x5F3759DF) - (i >> 1), jnp.float32)
    for _ in range(4):
        x = x * (1.5 - 0.5 * d * x * x)
    return x


def _splat16(ref, idx):
    return plsc.load_gather(ref, [jnp.full((LANES,), idx, jnp.int32)])


def _sc_degrees(dst2d):
    @functools.partial(
        pl.kernel,
        out_type=jax.ShapeDtypeStruct((2, NPAD), jnp.float32),
        mesh=_mesh(),
        compiler_params=_sc_params(),
        scratch_types=[
            pltpu.VMEM_SHARED((NPAD, 16), jnp.float32),
            pltpu.VMEM((8, 128), jnp.int32),
            pltpu.VMEM((128, 16), jnp.float32),
            pltpu.VMEM((128, 16), jnp.float32),
            pltpu.VMEM((128, 16), jnp.float32),
            pltpu.VMEM((128,), jnp.float32),
        ],
    )
    def deg_kernel(dst_hbm, dis_hbm, hist, idx_v, ones_v, zeros_v, hb_v, dis_v):
        c = lax.axis_index("c")
        s = lax.axis_index("s")
        rbase = s * ROWS_PT
        ebase = s * EROWS_PT

        _fill2d(ones_v, 128, 16, 1.0)
        _fill2d(zeros_v, 128, 16, 0.0)

        @pl.loop(0, RCH)
        def _(ch):
            pltpu.sync_copy(zeros_v, hist.at[pl.ds(rbase + ch * 128, 128)])

        plsc.subcore_barrier()

        @pl.loop(0, EB)
        def _(b):
            pltpu.sync_copy(dst_hbm.at[pl.ds(ebase + b * 8, 8)], idx_v)
            for j in range(8):
                pltpu.sync_copy(ones_v, hist.at[idx_v.at[j]], add=True)

        plsc.subcore_barrier()

        lane = lax.iota(jnp.int32, 16)

        @pl.loop(0, RCH)
        def _(ch):
            r0 = rbase + ch * 128
            pltpu.sync_copy(hist.at[pl.ds(r0, 128)], hb_v)
            for g in range(8):
                rows = jnp.full((LANES,), g * 16, jnp.int32) + lane
                cnt = plsc.load_gather(hb_v, [rows, jnp.zeros((LANES,), jnp.int32)])
                dis_v[pl.ds(g * 16, 16)] = _rsqrt16(cnt + 1.0)
            pltpu.sync_copy(dis_v, dis_hbm.at[c, pl.ds(r0, 128)])

    return deg_kernel(dst2d)


def _sc_propagate(src2d, dst2d, dis, h2):
    @functools.partial(
        pl.kernel,
        out_type=(
            jax.ShapeDtypeStruct((2, NPAD, DH), jnp.float32),
            jax.ShapeDtypeStruct((2, NPAD, DH), jnp.float32),
        ),
        mesh=_mesh(),
        compiler_params=_sc_params(),
        scratch_types=[
            pltpu.VMEM_SHARED((NPAD, DH), jnp.float32),
            pltpu.VMEM((8, 128), jnp.int32),
            pltpu.VMEM((8, 128), jnp.int32),
            pltpu.VMEM((8, 128), jnp.int32),
            pltpu.VMEM((8, 128), jnp.int32),
            pltpu.VMEM((128, DH), jnp.float32),
            pltpu.VMEM((128, DH), jnp.float32),
            pltpu.VMEM((128, DH), jnp.float32),
            pltpu.VMEM((128, DH), jnp.float32),
            pltpu.VMEM((128, DH), jnp.float32),
            pltpu.VMEM((128,), jnp.float32),
        ] + [pltpu.SemaphoreType.DMA] * 10,
    )
    def prop_kernel(src_hbm, dst_hbm, dis_hbm, h_hbm, y_hbm, z_hbm,
                    agg, is0, is1, id0, id1, g0, g1, g2, g3, zeros_v, dis_c,
                    gs0, gs1, gs2, gs3, ss0, ss1, ss2, ss3, es0, es1):
        ISRC = (is0, is1)
        IDST = (id0, id1)
        G = (g0, g1, g2, g3)
        GS = (gs0, gs1, gs2, gs3)
        SS = (ss0, ss1, ss2, ss3)
        ES = (es0, es1)
        a_v, y_v, h_v, o_v = g0, g1, g2, g3

        c = lax.axis_index("c")
        s = lax.axis_index("s")
        rbase = s * ROWS_PT
        ebase = s * EROWS_PT

        y_c = y_hbm.at[c]
        z_c = z_hbm.at[c]
        h_c = h_hbm.at[c]

        def wait_g(b):
            pltpu.make_async_copy(y_c.at[pl.ds(0, 128)], G[b], GS[b]).wait()

        def wait_s(b):
            if SYNC_SCATTER:
                return
            pltpu.make_async_copy(
                y_c.at[pl.ds(0, 128)], agg.at[pl.ds(0, 128)], SS[b]).wait()

        def load_idx(sg, p, sync):
            e0 = ebase + sg * 8
            if sync:
                pltpu.sync_copy(src_hbm.at[pl.ds(e0, 8)], ISRC[p])
                pltpu.sync_copy(dst_hbm.at[pl.ds(e0, 8)], IDST[p])
            else:
                pltpu.async_copy(src_hbm.at[pl.ds(e0, 8)], ISRC[p], ES[p])
                pltpu.async_copy(dst_hbm.at[pl.ds(e0, 8)], IDST[p], ES[p])

        def wait_idx(p):
            pltpu.make_async_copy(
                src_hbm.at[pl.ds(ebase, 8)], ISRC[p], ES[p]).wait()
            pltpu.make_async_copy(
                dst_hbm.at[pl.ds(ebase, 8)], IDST[p], ES[p]).wait()

        def gather(p, t, b):
            pltpu.async_copy(y_c.at[ISRC[p].at[t]], G[b], GS[b])

        SYNC_SCATTER = True

        def scatter(p, t, b):
            if SYNC_SCATTER:
                pltpu.sync_copy(G[b], agg.at[IDST[p].at[t]], add=True)
            else:
                pltpu.async_copy(G[b], agg.at[IDST[p].at[t]], SS[b], add=True)

        def process_sg(sg, p, first=False):
            pn = 1 - p
            for t in range(8):
                b = t % 4
                wait_g(b)
                scatter(p, t, b)
                t2 = t + 2
                b2 = t2 % 4
                if t == 2:
                    @pl.when(sg < EB - 1)
                    def _():
                        load_idx(sg + 1, pn, sync=False)
                if t2 < 8:
                    if first and t < 2:
                        gather(p, t2, b2)
                    else:
                        wait_s(b2)
                        gather(p, t2, b2)
                else:
                    @pl.when(sg < EB - 1)
                    def _():
                        wait_s(b2)
                        if t2 == 8:
                            wait_idx(pn)
                        gather(pn, t2 - 8, b2)

        def scatter_phase():
            load_idx(0, 0, sync=True)
            gather(0, 0, 0)
            gather(0, 1, 1)
            process_sg(0, 0, first=True)

            @pl.loop(0, (EB - 1) // 2)
            def _(m):
                process_sg(2 * m + 1, 1)
                process_sg(2 * m + 2, 0)

            for b in range(4):
                wait_s(b)

        _fill2d(zeros_v, 128, DH, 0.0)

        @pl.loop(0, RCH)
        def _(ch):
            pltpu.sync_copy(zeros_v, agg.at[pl.ds(rbase + ch * 128, 128)])

        @pl.loop(0, RCH)
        def _(ch):
            r0 = rbase + ch * 128
            pltpu.sync_copy(h_c.at[pl.ds(r0, 128)], h_v)
            pltpu.sync_copy(dis_hbm.at[c, pl.ds(r0, 128)], dis_c)

            @pl.loop(0, 128)
            def _(r):
                dv = _splat16(dis_c, r)
                for cc in range(DH // LANES):
                    sl = pl.ds(cc * LANES, LANES)
                    o_v[r, sl] = dv * h_v[r, sl]

            pltpu.sync_copy(o_v, y_c.at[pl.ds(r0, 128)])

        plsc.subcore_barrier()

        def dense_phase(last):
            @pl.loop(0, RCH)
            def _(ch):
                r0 = rbase + ch * 128
                pltpu.sync_copy(agg.at[pl.ds(r0, 128)], a_v)
                pltpu.sync_copy(zeros_v, agg.at[pl.ds(r0, 128)])
                pltpu.sync_copy(y_c.at[pl.ds(r0, 128)], y_v)
                pltpu.sync_copy(h_c.at[pl.ds(r0, 128)], h_v)
                pltpu.sync_copy(dis_hbm.at[c, pl.ds(r0, 128)], dis_c)

                @pl.loop(0, 128)
                def _(r):
                    dv = _splat16(dis_c, r)
                    c09 = (1.0 - ALPHA) * dv
                    for cc in range(DH // LANES):
                        sl = pl.ds(cc * LANES, LANES)
                        v = a_v[r, sl] + y_v[r, sl]
                        z = c09 * v + ALPHA * h_v[r, sl]
                        o_v[r, sl] = z if last else dv * z

                if last:
                    pltpu.sync_copy(o_v, z_c.at[pl.ds(r0, 128)])
                else:
                    pltpu.sync_copy(o_v, y_c.at[pl.ds(r0, 128)])

        @pl.loop(0, K - 1)
        def _(k):
            scatter_phase()
            plsc.subcore_barrier()
            dense_phase(False)
            plsc.subcore_barrier()

        scatter_phase()
        plsc.subcore_barrier()
        dense_phase(True)

    return prop_kernel(src2d, dst2d, dis, h2)


_MLP_BLK = 1024


def _mlp_body(x_ref, w1_ref, b1_ref, w2_ref, b2_ref, o_ref):
    hid = jnp.dot(x_ref[...], w1_ref[...], preferred_element_type=jnp.float32)
    hid = jnp.maximum(hid + b1_ref[...], 0.0)
    out = jnp.dot(hid, w2_ref[...], preferred_element_type=jnp.float32)
    o_ref[...] = out + b2_ref[...]


def _tc_mlp(xp, W1, b1, W2, b2):
    grid = NPAD // _MLP_BLK
    return pl.pallas_call(
        _mlp_body,
        grid=(grid,),
        in_specs=[
            pl.BlockSpec((_MLP_BLK, D_IN), lambda i: (i, 0)),
            pl.BlockSpec((D_IN, D_IN), lambda i: (0, 0)),
            pl.BlockSpec((1, D_IN), lambda i: (0, 0)),
            pl.BlockSpec((D_IN, D_OUT), lambda i: (0, 0)),
            pl.BlockSpec((1, D_OUT), lambda i: (0, 0)),
        ],
        out_specs=pl.BlockSpec((_MLP_BLK, D_OUT), lambda i: (i, 0)),
        out_shape=jax.ShapeDtypeStruct((NPAD, D_OUT), jnp.float32),
    )(xp, W1, b1.reshape(1, D_IN), W2, b2.reshape(1, D_OUT))


def kernel(x, edge_index, W1, b1, W2, b2):
    src = edge_index[0]
    dst = edge_index[1]
    pad = jnp.full((EPAD - E,), N, jnp.int32)
    src2d = jnp.concatenate([src, pad]).reshape(EROWS, 128)
    dst2d = jnp.concatenate([dst, pad]).reshape(EROWS, 128)
    xp = jnp.pad(x, ((0, NPAD - N), (0, 0)))

    h = _tc_mlp(xp, W1, b1, W2, b2)
    h2 = jnp.stack([h[:, :DH], h[:, DH:]])
    dis = _sc_degrees(dst2d)
    _, z2 = _sc_propagate(src2d, dst2d, dis, h2)
    return jnp.concatenate([z2[0, :N], z2[1, :N]], axis=1)


# speedup vs baseline: 18.7864x; 1.0758x over previous
import dataclasses
import functools

import jax
import jax.numpy as jnp
from jax import lax
from jax.experimental import pallas as pl
from jax.experimental.pallas import tpu as pltpu
from jax.experimental.pallas import tpu_sc as plsc

N = 50000
E = 800000
D_IN = 128
D_OUT = 64
DH = 32
K = 10
ALPHA = 0.1

NTILES = 16
LANES = 16

RCH = 25
ROWS_PT = RCH * 128
NPAD = NTILES * ROWS_PT

EB = 49
EROWS_PT = EB * 8
EROWS = NTILES * EROWS_PT
EPAD = EROWS * 128

_mesh = lambda: plsc.VectorSubcoreMesh(core_axis_name="c", subcore_axis_name="s")


def _sc_params():
    cp = pltpu.CompilerParams()
    fields = pltpu.CompilerParams.__dataclass_fields__
    if "needs_layout_passes" in fields:
        cp = dataclasses.replace(cp, needs_layout_passes=False)
    if "use_tc_tiling_on_sc" in fields:
        cp = dataclasses.replace(cp, use_tc_tiling_on_sc=False)
    return cp


def _fill2d(ref, nrows, ncols, val):
    v = jnp.full((LANES,), val, jnp.float32)

    @pl.loop(0, nrows)
    def _(r):
        for j in range(ncols // LANES):
            ref[r, pl.ds(j * LANES, LANES)] = v


def _rsqrt16(d):
    i = plsc.bitcast(d, jnp.int32)
    x = plsc.bitcast(jnp.int32(0x5F3759DF) - (i >> 1), jnp.float32)
    for _ in range(4):
        x = x * (1.5 - 0.5 * d * x * x)
    return x


def _splat16(ref, idx):
    return plsc.load_gather(ref, [jnp.full((LANES,), idx, jnp.int32)])


def _sc_degrees(dst2d):
    @functools.partial(
        pl.kernel,
        out_type=jax.ShapeDtypeStruct((2, NPAD), jnp.float32),
        mesh=_mesh(),
        compiler_params=_sc_params(),
        scratch_types=[
            pltpu.VMEM_SHARED((NPAD, 16), jnp.float32),
            pltpu.VMEM((8, 128), jnp.int32),
            pltpu.VMEM((128, 16), jnp.float32),
            pltpu.VMEM((128, 16), jnp.float32),
            pltpu.VMEM((128, 16), jnp.float32),
            pltpu.VMEM((128,), jnp.float32),
        ],
    )
    def deg_kernel(dst_hbm, dis_hbm, hist, idx_v, ones_v, zeros_v, hb_v, dis_v):
        c = lax.axis_index("c")
        s = lax.axis_index("s")
        rbase = s * ROWS_PT
        ebase = s * EROWS_PT

        _fill2d(ones_v, 128, 16, 1.0)
        _fill2d(zeros_v, 128, 16, 0.0)

        @pl.loop(0, RCH)
        def _(ch):
            pltpu.sync_copy(zeros_v, hist.at[pl.ds(rbase + ch * 128, 128)])

        plsc.subcore_barrier()

        @pl.loop(0, EB)
        def _(b):
            pltpu.sync_copy(dst_hbm.at[pl.ds(ebase + b * 8, 8)], idx_v)
            for j in range(8):
                pltpu.sync_copy(ones_v, hist.at[idx_v.at[j]], add=True)

        plsc.subcore_barrier()

        lane = lax.iota(jnp.int32, 16)

        @pl.loop(0, RCH)
        def _(ch):
            r0 = rbase + ch * 128
            pltpu.sync_copy(hist.at[pl.ds(r0, 128)], hb_v)
            for g in range(8):
                rows = jnp.full((LANES,), g * 16, jnp.int32) + lane
                cnt = plsc.load_gather(hb_v, [rows, jnp.zeros((LANES,), jnp.int32)])
                dis_v[pl.ds(g * 16, 16)] = _rsqrt16(cnt + 1.0)
            pltpu.sync_copy(dis_v, dis_hbm.at[c, pl.ds(r0, 128)])

    return deg_kernel(dst2d)


def _sc_propagate(src2d, dst2d, dis, h2):
    @functools.partial(
        pl.kernel,
        out_type=(
            jax.ShapeDtypeStruct((2, NPAD, DH), jnp.float32),
            jax.ShapeDtypeStruct((2, NPAD, DH), jnp.float32),
        ),
        mesh=_mesh(),
        compiler_params=_sc_params(),
        scratch_types=[
            pltpu.VMEM_SHARED((NPAD, DH), jnp.float32),
            pltpu.VMEM((8, 128), jnp.int32),
            pltpu.VMEM((8, 128), jnp.int32),
            pltpu.VMEM((8, 128), jnp.int32),
            pltpu.VMEM((8, 128), jnp.int32),
            pltpu.VMEM((128, DH), jnp.float32),
            pltpu.VMEM((128, DH), jnp.float32),
            pltpu.VMEM((128, DH), jnp.float32),
            pltpu.VMEM((128, DH), jnp.float32),
            pltpu.VMEM((128, DH), jnp.float32),
            pltpu.VMEM((128,), jnp.float32),
        ] + [pltpu.SemaphoreType.DMA] * 10,
    )
    def prop_kernel(src_hbm, dst_hbm, dis_hbm, h_hbm, y_hbm, z_hbm,
                    agg, is0, is1, id0, id1, g0, g1, g2, g3, zeros_v, dis_c,
                    gs0, gs1, gs2, gs3, ss0, ss1, ss2, ss3, es0, es1):
        ISRC = (is0, is1)
        IDST = (id0, id1)
        G = (g0, g1, g2, g3)
        GS = (gs0, gs1, gs2, gs3)
        SS = (ss0, ss1, ss2, ss3)
        ES = (es0, es1)
        a_v, y_v, h_v, o_v = g0, g1, g2, g3

        c = lax.axis_index("c")
        s = lax.axis_index("s")
        rbase = s * ROWS_PT
        ebase = s * EROWS_PT

        y_c = y_hbm.at[c]
        z_c = z_hbm.at[c]
        h_c = h_hbm.at[c]

        def wait_g(b):
            pltpu.make_async_copy(y_c.at[is0.at[0]], G[b], GS[b]).wait()

        def wait_s(b):
            pltpu.make_async_copy(G[b], agg.at[id0.at[0]], SS[b]).wait()

        def load_idx(sg, p, sync):
            e0 = ebase + sg * 8
            if sync:
                pltpu.sync_copy(src_hbm.at[pl.ds(e0, 8)], ISRC[p])
                pltpu.sync_copy(dst_hbm.at[pl.ds(e0, 8)], IDST[p])
            else:
                pltpu.async_copy(src_hbm.at[pl.ds(e0, 8)], ISRC[p], ES[p])
                pltpu.async_copy(dst_hbm.at[pl.ds(e0, 8)], IDST[p], ES[p])

        def wait_idx(p):
            pltpu.make_async_copy(
                src_hbm.at[pl.ds(ebase, 8)], ISRC[p], ES[p]).wait()
            pltpu.make_async_copy(
                dst_hbm.at[pl.ds(ebase, 8)], IDST[p], ES[p]).wait()

        def gather(p, t, b):
            pltpu.async_copy(y_c.at[ISRC[p].at[t]], G[b], GS[b])

        def scatter(p, t, b):
            pltpu.async_copy(G[b], agg.at[IDST[p].at[t]], SS[b], add=True)

        def process_sg(sg, p, first=False):
            pn = 1 - p
            for t in range(8):
                b = t % 4
                wait_g(b)
                scatter(p, t, b)
                t2 = t + 2
                b2 = t2 % 4
                if t == 2:
                    @pl.when(sg < EB - 1)
                    def _():
                        load_idx(sg + 1, pn, sync=False)
                if t2 < 8:
                    if first and t < 2:
                        gather(p, t2, b2)
                    else:
                        wait_s(b2)
                        gather(p, t2, b2)
                else:
                    @pl.when(sg < EB - 1)
                    def _():
                        wait_s(b2)
                        if t2 == 8:
                            wait_idx(pn)
                        gather(pn, t2 - 8, b2)

        def scatter_phase():
            load_idx(0, 0, sync=True)
            gather(0, 0, 0)
            gather(0, 1, 1)
            process_sg(0, 0, first=True)

            @pl.loop(0, (EB - 1) // 2)
            def _(m):
                process_sg(2 * m + 1, 1)
                process_sg(2 * m + 2, 0)

            for b in range(4):
                wait_s(b)

        _fill2d(zeros_v, 128, DH, 0.0)

        @pl.loop(0, RCH)
        def _(ch):
            pltpu.sync_copy(zeros_v, agg.at[pl.ds(rbase + ch * 128, 128)])

        @pl.loop(0, RCH)
        def _(ch):
            r0 = rbase + ch * 128
            pltpu.sync_copy(h_c.at[pl.ds(r0, 128)], h_v)
            pltpu.sync_copy(dis_hbm.at[c, pl.ds(r0, 128)], dis_c)

            @pl.loop(0, 128)
            def _(r):
                dv = _splat16(dis_c, r)
                for cc in range(DH // LANES):
                    sl = pl.ds(cc * LANES, LANES)
                    o_v[r, sl] = dv * h_v[r, sl]

            pltpu.sync_copy(o_v, y_c.at[pl.ds(r0, 128)])

        plsc.subcore_barrier()

        def dense_phase(last):
            @pl.loop(0, RCH)
            def _(ch):
                r0 = rbase + ch * 128
                pltpu.sync_copy(agg.at[pl.ds(r0, 128)], a_v)
                pltpu.sync_copy(zeros_v, agg.at[pl.ds(r0, 128)])
                pltpu.sync_copy(y_c.at[pl.ds(r0, 128)], y_v)
                pltpu.sync_copy(h_c.at[pl.ds(r0, 128)], h_v)
                pltpu.sync_copy(dis_hbm.at[c, pl.ds(r0, 128)], dis_c)

                @pl.loop(0, 128)
                def _(r):
                    dv = _splat16(dis_c, r)
                    c09 = (1.0 - ALPHA) * dv
                    for cc in range(DH // LANES):
                        sl = pl.ds(cc * LANES, LANES)
                        v = a_v[r, sl] + y_v[r, sl]
                        z = c09 * v + ALPHA * h_v[r, sl]
                        o_v[r, sl] = z if last else dv * z

                if last:
                    pltpu.sync_copy(o_v, z_c.at[pl.ds(r0, 128)])
                else:
                    pltpu.sync_copy(o_v, y_c.at[pl.ds(r0, 128)])

        @pl.loop(0, K - 1)
        def _(k):
            scatter_phase()
            plsc.subcore_barrier()
            dense_phase(False)
            plsc.subcore_barrier()

        scatter_phase()
        plsc.subcore_barrier()
        dense_phase(True)

    return prop_kernel(src2d, dst2d, dis, h2)


_MLP_BLK = 1024


def _mlp_body(x_ref, w1_ref, b1_ref, w2_ref, b2_ref, o_ref):
    hid = jnp.dot(x_ref[...], w1_ref[...], preferred_element_type=jnp.float32)
    hid = jnp.maximum(hid + b1_ref[...], 0.0)
    out = jnp.dot(hid, w2_ref[...], preferred_element_type=jnp.float32)
    o_ref[...] = out + b2_ref[...]


def _tc_mlp(xp, W1, b1, W2, b2):
    grid = NPAD // _MLP_BLK
    return pl.pallas_call(
        _mlp_body,
        grid=(grid,),
        in_specs=[
            pl.BlockSpec((_MLP_BLK, D_IN), lambda i: (i, 0)),
            pl.BlockSpec((D_IN, D_IN), lambda i: (0, 0)),
            pl.BlockSpec((1, D_IN), lambda i: (0, 0)),
            pl.BlockSpec((D_IN, D_OUT), lambda i: (0, 0)),
            pl.BlockSpec((1, D_OUT), lambda i: (0, 0)),
        ],
        out_specs=pl.BlockSpec((_MLP_BLK, D_OUT), lambda i: (i, 0)),
        out_shape=jax.ShapeDtypeStruct((NPAD, D_OUT), jnp.float32),
    )(xp, W1, b1.reshape(1, D_IN), W2, b2.reshape(1, D_OUT))


def kernel(x, edge_index, W1, b1, W2, b2):
    src = edge_index[0]
    dst = edge_index[1]
    pad = jnp.full((EPAD - E,), N, jnp.int32)
    src2d = jnp.concatenate([src, pad]).reshape(EROWS, 128)
    dst2d = jnp.concatenate([dst, pad]).reshape(EROWS, 128)
    xp = jnp.pad(x, ((0, NPAD - N), (0, 0)))

    h = _tc_mlp(xp, W1, b1, W2, b2)
    h2 = jnp.stack([h[:, :DH], h[:, DH:]])
    dis = _sc_degrees(dst2d)
    _, z2 = _sc_propagate(src2d, dst2d, dis, h2)
    return jnp.concatenate([z2[0, :N], z2[1, :N]], axis=1)


# speedup vs baseline: 21.1606x; 1.1264x over previous
import dataclasses
import functools

import jax
import jax.numpy as jnp
from jax import lax
from jax.experimental import pallas as pl
from jax.experimental.pallas import tpu as pltpu
from jax.experimental.pallas import tpu_sc as plsc

N = 50000
E = 800000
D_IN = 128
D_OUT = 64
DH = 32
K = 10
ALPHA = 0.1

NTILES = 16
LANES = 16

RCH = 25
ROWS_PT = RCH * 128
NPAD = NTILES * ROWS_PT

EB = 49
EROWS_PT = EB * 8
EROWS = NTILES * EROWS_PT
EPAD = EROWS * 128

_mesh = lambda: plsc.VectorSubcoreMesh(core_axis_name="c", subcore_axis_name="s")


def _sc_params():
    cp = pltpu.CompilerParams()
    fields = pltpu.CompilerParams.__dataclass_fields__
    if "needs_layout_passes" in fields:
        cp = dataclasses.replace(cp, needs_layout_passes=False)
    if "use_tc_tiling_on_sc" in fields:
        cp = dataclasses.replace(cp, use_tc_tiling_on_sc=False)
    return cp


def _fill2d(ref, nrows, ncols, val):
    v = jnp.full((LANES,), val, jnp.float32)

    @pl.loop(0, nrows)
    def _(r):
        for j in range(ncols // LANES):
            ref[r, pl.ds(j * LANES, LANES)] = v


def _rsqrt16(d):
    i = plsc.bitcast(d, jnp.int32)
    x = plsc.bitcast(jnp.int32(0x5F3759DF) - (i >> 1), jnp.float32)
    for _ in range(4):
        x = x * (1.5 - 0.5 * d * x * x)
    return x


def _splat16(ref, idx):
    return plsc.load_gather(ref, [jnp.full((LANES,), idx, jnp.int32)])


def _sc_degrees(dst2d):
    @functools.partial(
        pl.kernel,
        out_type=jax.ShapeDtypeStruct((2, NPAD), jnp.float32),
        mesh=_mesh(),
        compiler_params=_sc_params(),
        scratch_types=[
            pltpu.VMEM_SHARED((NPAD, 16), jnp.float32),
            pltpu.VMEM((8, 128), jnp.int32),
            pltpu.VMEM((128, 16), jnp.float32),
            pltpu.VMEM((128, 16), jnp.float32),
            pltpu.VMEM((128, 16), jnp.float32),
            pltpu.VMEM((128,), jnp.float32),
        ],
    )
    def deg_kernel(dst_hbm, dis_hbm, hist, idx_v, ones_v, zeros_v, hb_v, dis_v):
        c = lax.axis_index("c")
        s = lax.axis_index("s")
        rbase = s * ROWS_PT
        ebase = s * EROWS_PT

        _fill2d(ones_v, 128, 16, 1.0)
        _fill2d(zeros_v, 128, 16, 0.0)

        @pl.loop(0, RCH)
        def _(ch):
            pltpu.sync_copy(zeros_v, hist.at[pl.ds(rbase + ch * 128, 128)])

        plsc.subcore_barrier()

        @pl.loop(0, EB)
        def _(b):
            pltpu.sync_copy(dst_hbm.at[pl.ds(ebase + b * 8, 8)], idx_v)
            for j in range(8):
                pltpu.sync_copy(ones_v, hist.at[idx_v.at[j]], add=True)

        plsc.subcore_barrier()

        lane = lax.iota(jnp.int32, 16)

        @pl.loop(0, RCH)
        def _(ch):
            r0 = rbase + ch * 128
            pltpu.sync_copy(hist.at[pl.ds(r0, 128)], hb_v)
            for g in range(8):
                rows = jnp.full((LANES,), g * 16, jnp.int32) + lane
                cnt = plsc.load_gather(hb_v, [rows, jnp.zeros((LANES,), jnp.int32)])
                dis_v[pl.ds(g * 16, 16)] = _rsqrt16(cnt + 1.0)
            pltpu.sync_copy(dis_v, dis_hbm.at[c, pl.ds(r0, 128)])

    return deg_kernel(dst2d)


def _sc_propagate(src2d, dst2d, dis, h2):
    @functools.partial(
        pl.kernel,
        out_type=(
            jax.ShapeDtypeStruct((2, NPAD, DH), jnp.float32),
            jax.ShapeDtypeStruct((2, NPAD, DH), jnp.float32),
        ),
        mesh=_mesh(),
        compiler_params=_sc_params(),
        scratch_types=[
            pltpu.VMEM_SHARED((NPAD, DH), jnp.float32),
            pltpu.VMEM((8, 128), jnp.int32),
            pltpu.VMEM((8, 128), jnp.int32),
            pltpu.VMEM((8, 128), jnp.int32),
            pltpu.VMEM((8, 128), jnp.int32),
            pltpu.VMEM((128, DH), jnp.float32),
            pltpu.VMEM((128, DH), jnp.float32),
            pltpu.VMEM((128, DH), jnp.float32),
            pltpu.VMEM((128, DH), jnp.float32),
            pltpu.VMEM((128, DH), jnp.float32),
            pltpu.VMEM((128,), jnp.float32),
        ] + [pltpu.SemaphoreType.DMA] * 10,
    )
    def prop_kernel(src_hbm, dst_hbm, dis_hbm, h_hbm, y_hbm, z_hbm,
                    agg, is0, is1, id0, id1, g0, g1, g2, g3, zeros_v, dis_c,
                    gs0, gs1, gs2, gs3, ss0, ss1, ss2, ss3, es0, es1):
        ISRC = (is0, is1)
        IDST = (id0, id1)
        G = (g0, g1, g2, g3)
        GS = (gs0, gs1, gs2, gs3)
        SS = (ss0, ss1, ss2, ss3)
        ES = (es0, es1)
        a_v, y_v, h_v, o_v = g0, g1, g2, g3

        c = lax.axis_index("c")
        s = lax.axis_index("s")
        rbase = s * ROWS_PT
        ebase = s * EROWS_PT

        y_c = y_hbm.at[c]
        z_c = z_hbm.at[c]
        h_c = h_hbm.at[c]

        def wait_g(b):
            pltpu.make_async_copy(y_c.at[is0.at[0]], G[b], GS[b]).wait()

        def wait_s(b):
            pltpu.make_async_copy(G[b], agg.at[id0.at[0]], SS[b]).wait()

        def load_idx(sg, p, sync):
            e0 = ebase + sg * 8
            if sync:
                pltpu.sync_copy(src_hbm.at[pl.ds(e0, 8)], ISRC[p])
                pltpu.sync_copy(dst_hbm.at[pl.ds(e0, 8)], IDST[p])
            else:
                pltpu.async_copy(src_hbm.at[pl.ds(e0, 8)], ISRC[p], ES[p])
                pltpu.async_copy(dst_hbm.at[pl.ds(e0, 8)], IDST[p], ES[p])

        def wait_idx(p):
            pltpu.make_async_copy(
                src_hbm.at[pl.ds(ebase, 8)], ISRC[p], ES[p]).wait()
            pltpu.make_async_copy(
                dst_hbm.at[pl.ds(ebase, 8)], IDST[p], ES[p]).wait()

        def gather(p, t, b):
            pltpu.async_copy(y_c.at[ISRC[p].at[t]], G[b], GS[b])

        def scatter(p, t, b):
            pltpu.async_copy(G[b], agg.at[IDST[p].at[t]], SS[b], add=True)

        def process_sg(sg, p, first=False):
            pn = 1 - p
            for t in range(8):
                b = t % 4
                wait_g(b)
                scatter(p, t, b)
                t2 = t + 3
                b2 = t2 % 4
                if t == 2:
                    @pl.when(sg < EB - 1)
                    def _():
                        load_idx(sg + 1, pn, sync=False)
                if t2 < 8:
                    if first and t < 1:
                        gather(p, t2, b2)
                    else:
                        wait_s(b2)
                        gather(p, t2, b2)
                else:
                    @pl.when(sg < EB - 1)
                    def _():
                        wait_s(b2)
                        if t2 == 8:
                            wait_idx(pn)
                        gather(pn, t2 - 8, b2)

        def scatter_phase():
            load_idx(0, 0, sync=True)
            gather(0, 0, 0)
            gather(0, 1, 1)
            gather(0, 2, 2)
            process_sg(0, 0, first=True)

            @pl.loop(0, (EB - 1) // 2)
            def _(m):
                process_sg(2 * m + 1, 1)
                process_sg(2 * m + 2, 0)

            for b in range(4):
                wait_s(b)

        _fill2d(zeros_v, 128, DH, 0.0)

        @pl.loop(0, RCH)
        def _(ch):
            pltpu.sync_copy(zeros_v, agg.at[pl.ds(rbase + ch * 128, 128)])

        @pl.loop(0, RCH)
        def _(ch):
            r0 = rbase + ch * 128
            pltpu.sync_copy(h_c.at[pl.ds(r0, 128)], h_v)
            pltpu.sync_copy(dis_hbm.at[c, pl.ds(r0, 128)], dis_c)

            @pl.loop(0, 128)
            def _(r):
                dv = _splat16(dis_c, r)
                for cc in range(DH // LANES):
                    sl = pl.ds(cc * LANES, LANES)
                    o_v[r, sl] = dv * h_v[r, sl]

            pltpu.sync_copy(o_v, y_c.at[pl.ds(r0, 128)])

        plsc.subcore_barrier()

        def dense_phase(last):
            @pl.loop(0, RCH)
            def _(ch):
                r0 = rbase + ch * 128
                pltpu.sync_copy(agg.at[pl.ds(r0, 128)], a_v)
                pltpu.sync_copy(zeros_v, agg.at[pl.ds(r0, 128)])
                pltpu.sync_copy(y_c.at[pl.ds(r0, 128)], y_v)
                pltpu.sync_copy(h_c.at[pl.ds(r0, 128)], h_v)
                pltpu.sync_copy(dis_hbm.at[c, pl.ds(r0, 128)], dis_c)

                @pl.loop(0, 128)
                def _(r):
                    dv = _splat16(dis_c, r)
                    c09 = (1.0 - ALPHA) * dv
                    for cc in range(DH // LANES):
                        sl = pl.ds(cc * LANES, LANES)
                        v = a_v[r, sl] + y_v[r, sl]
                        z = c09 * v + ALPHA * h_v[r, sl]
                        o_v[r, sl] = z if last else dv * z

                if last:
                    pltpu.sync_copy(o_v, z_c.at[pl.ds(r0, 128)])
                else:
                    pltpu.sync_copy(o_v, y_c.at[pl.ds(r0, 128)])

        @pl.loop(0, K - 1)
        def _(k):
            scatter_phase()
            plsc.subcore_barrier()
            dense_phase(False)
            plsc.subcore_barrier()

        scatter_phase()
        plsc.subcore_barrier()
        dense_phase(True)

    return prop_kernel(src2d, dst2d, dis, h2)


_MLP_BLK = 1024


def _mlp_body(x_ref, w1_ref, b1_ref, w2_ref, b2_ref, o_ref):
    hid = jnp.dot(x_ref[...], w1_ref[...], preferred_element_type=jnp.float32)
    hid = jnp.maximum(hid + b1_ref[...], 0.0)
    out = jnp.dot(hid, w2_ref[...], preferred_element_type=jnp.float32)
    o_ref[...] = out + b2_ref[...]


def _tc_mlp(xp, W1, b1, W2, b2):
    grid = NPAD // _MLP_BLK
    return pl.pallas_call(
        _mlp_body,
        grid=(grid,),
        in_specs=[
            pl.BlockSpec((_MLP_BLK, D_IN), lambda i: (i, 0)),
            pl.BlockSpec((D_IN, D_IN), lambda i: (0, 0)),
            pl.BlockSpec((1, D_IN), lambda i: (0, 0)),
            pl.BlockSpec((D_IN, D_OUT), lambda i: (0, 0)),
            pl.BlockSpec((1, D_OUT), lambda i: (0, 0)),
        ],
        out_specs=pl.BlockSpec((_MLP_BLK, D_OUT), lambda i: (i, 0)),
        out_shape=jax.ShapeDtypeStruct((NPAD, D_OUT), jnp.float32),
    )(xp, W1, b1.reshape(1, D_IN), W2, b2.reshape(1, D_OUT))


def kernel(x, edge_index, W1, b1, W2, b2):
    src = edge_index[0]
    dst = edge_index[1]
    pad = jnp.full((EPAD - E,), N, jnp.int32)
    src2d = jnp.concatenate([src, pad]).reshape(EROWS, 128)
    dst2d = jnp.concatenate([dst, pad]).reshape(EROWS, 128)
    xp = jnp.pad(x, ((0, NPAD - N), (0, 0)))

    h = _tc_mlp(xp, W1, b1, W2, b2)
    h2 = jnp.stack([h[:, :DH], h[:, DH:]])
    dis = _sc_degrees(dst2d)
    _, z2 = _sc_propagate(src2d, dst2d, dis, h2)
    return jnp.concatenate([z2[0, :N], z2[1, :N]], axis=1)
